# baseline (speedup 1.0000x reference)
import functools

import jax
import jax.numpy as jnp
from jax import lax
from jax.experimental import pallas as pl
from jax.experimental.pallas import tpu as pltpu
from jax.experimental.pallas import tpu_sc as plsc

N_NODES = 10000
N_EDGES = 320000
EV = N_EDGES // 8
NV = N_NODES // 8
BLK = 5000
H = 16

_f32 = jnp.float32


def _kron8(W):
    return jnp.kron(jnp.eye(8, dtype=_f32), W.astype(_f32))


def _tile8(v):
    return jnp.tile(jnp.reshape(v, (1, H)), (1, 8))


def _dot(a, b):
    return jnp.dot(a, b, preferred_element_type=_f32)


def _prep_edges_body(e_ref, ke1, be1, ke2, be2, k0, b1e, eh0_ref, e0_ref):
    e = e_ref[:]
    h = jnp.maximum(_dot(e, ke1[:]) + be1[:], 0.0)
    eh0_ref[:] = _dot(h, ke2[:]) + be2[:]
    e0_ref[:] = _dot(e, k0[:]) + b1e[:]


def _prep_edges(ev, params):
    (W1, b1), (W2, b2) = params['enc']['e']
    W1r, b1r = params['rec']['e'][0]
    big = pl.BlockSpec((BLK, 128), lambda i: (i, 0))
    w = pl.BlockSpec((128, 128), lambda i: (0, 0))
    s = pl.BlockSpec((1, 128), lambda i: (0, 0))
    return pl.pallas_call(
        _prep_edges_body,
        grid=(EV // BLK,),
        in_specs=[big, w, s, w, s, w, s],
        out_specs=[big, big],
        out_shape=[jax.ShapeDtypeStruct((EV, 128), _f32)] * 2,
    )(ev, _kron8(W1), _tile8(b1), _kron8(W2), _tile8(b2),
      _kron8(W1r[0:16]), _tile8(b1r))


def _prep_nodes_body(x_ref, u_ref, w1x, b1x, w2x, b2x, wxp, wxn,
                     wxp1, w1u, b1u, w2u, b2u, ft,
                     xh0_ref, xp0_ref, xp1_ref, xn0_ref, uh0_ref):
    x = x_ref[:]
    h = jnp.maximum(_dot(x, w1x[:]) + b1x[:], 0.0)
    xh0 = _dot(h, w2x[:]) + b2x[:]
    xh0_ref[:] = xh0
    xp0 = _dot(x, wxp[:])
    xp0_ref[:] = xp0
    xp1_ref[:] = xp0 + _dot(xh0, wxp1[:])
    xn0_ref[:] = _dot(x, wxn[:])
    u16 = u_ref[:, :H]
    hu = jnp.maximum(_dot(u16, w1u[:]) + b1u[:], 0.0)
    uh0_ref[:] = _dot(_dot(hu, w2u[:]) + b2u[:], ft[:])


def _prep_nodes(x, u_t, params):
    (W1x, b1x), (W2x, b2x) = params['enc']['x']
    (W1u, b1u), (W2u, b2u) = params['enc']['u']
    W1r, _ = params['rec']['e'][0]
    W1n, _ = params['rec']['x'][0]
    ft = jnp.tile(jnp.eye(H, dtype=_f32), (1, 8))
    outs = pl.pallas_call(
        _prep_nodes_body,
        out_shape=[jax.ShapeDtypeStruct((N_NODES, H), _f32)] * 4
        + [jax.ShapeDtypeStruct((1, 128), _f32)],
    )(x, u_t, W1x, jnp.reshape(b1x, (1, H)), W2x,
      jnp.reshape(b2x, (1, H)), W1r[32:160], W1n[0:128], W1r[160:176],
      W1u, jnp.reshape(b1u, (1, H)), W2u, jnp.reshape(b2u, (1, H)), ft)
    return outs


def _edge_mlp_body(e0_ref, eh_ref, dx_ref, sx_ref, u_t, s_t, ku, ks, k16,
                   k2, b2, out_ref):
    ut = _dot(u_t[:], ku[:]) + _dot(s_t[:], ks[:])
    h1 = jnp.maximum(e0_ref[:] + _dot(eh_ref[:], k16[:])
                     + (dx_ref[:] - sx_ref[:]) + ut, 0.0)
    out_ref[:] = _dot(h1, k2[:]) + b2[:]


def _rec_edge(e0v, ehv, dxv, sxv, u_t, s_t, params):
    (W1, _), (W2, b2) = params['rec']['e']
    big = pl.BlockSpec((BLK, 128), lambda i: (i, 0))
    w = pl.BlockSpec((128, 128), lambda i: (0, 0))
    s = pl.BlockSpec((1, 128), lambda i: (0, 0))
    return pl.pallas_call(
        _edge_mlp_body,
        grid=(EV // BLK,),
        in_specs=[big, big, big, big, s, s, w, w, w, w, s],
        out_specs=big,
        out_shape=jax.ShapeDtypeStruct((EV, 128), _f32),
    )(e0v, ehv, dxv, sxv, u_t, s_t, _kron8(W1[176:192]),
      _kron8(W1[192:208]), _kron8(W1[16:32]), _kron8(W2), _tile8(b2))


def _rec_node_body(p0, p1, c0, c1, xn0, xh, u_t, s_t, xp0n, kb, kc, kun, ksn,
                   b1n, k2n, b2n, w1u, b1u, w2u, b2u, kxn, f, ft,
                   xnew_ref, xpn_ref, unew_ref):
    seg = p0[:] + p1[:]
    agg = seg / jnp.maximum(c0[:] + c1[:], 1.0)
    un = _dot(u_t[:], kun[:]) + _dot(s_t[:], ksn[:])
    h1 = jnp.maximum(xn0[:] + _dot(xh[:], kb[:]) + _dot(agg, kc[:]) + un
                     + b1n[:], 0.0)
    xnew = _dot(h1, k2n[:]) + b2n[:]
    xnew_ref[:] = xnew
    xpn_ref[:] = xp0n[:] + _dot(xnew, kxn[:])
    xa = _dot(jnp.sum(xnew, axis=0, keepdims=True), f[:]) * (1.0 / N_NODES)
    ea = _dot(jnp.sum(seg, axis=0, keepdims=True), f[:]) * (1.0 / N_EDGES)
    ucat = jnp.concatenate([u_t[:, :H], s_t[:, :H], xa, ea], axis=1)
    hu = jnp.maximum(_dot(ucat, w1u[:]) + b1u[:, :H], 0.0)
    unew_ref[:] = _dot(_dot(hu, w2u[:]) + b2u[:, :H], ft[:])


def _rec_node(parts, cnt, xn0v, xhv, u_t, s_t, xp0nextv, kxnext, params):
    (W1n, b1n), (W2n, b2n) = params['rec']['x']
    (W1u, b1u), (W2u, b2u) = params['rec']['u']
    f = jnp.tile(jnp.eye(H, dtype=_f32), (8, 1))
    ft = f.T
    return pl.pallas_call(
        _rec_node_body,
        out_shape=[jax.ShapeDtypeStruct((NV, 128), _f32),
                   jax.ShapeDtypeStruct((NV, 128), _f32),
                   jax.ShapeDtypeStruct((1, 128), _f32)],
    )(parts[0], parts[1], cnt[0], cnt[1], xn0v, xhv, u_t, s_t, xp0nextv,
      _kron8(W1n[128:144]), _kron8(W1n[144:160]), _kron8(W1n[160:176]),
      _kron8(W1n[176:192]), _tile8(b1n), _kron8(W2n), _tile8(b2n),
      W1u, _tile8(b1u), W2u, _tile8(b2u), _kron8(kxnext), f, ft)


def _att_edge_body(eh_ref, dx_ref, sx_ref, uh_t, ku, k0, k2, b1, b2,
                   ea_ref, e2_ref, e2s_ref):
    ut = _dot(uh_t[:], ku[:]) + b1[:]
    eh = eh_ref[:]
    h1 = jnp.maximum(_dot(eh, k0[:]) + (dx_ref[:] - sx_ref[:]) + ut, 0.0)
    ea = _dot(h1, k2[:]) + b2[:]
    ea_ref[:] = ea
    e2 = ea * eh
    e2_ref[:] = e2

    @pl.when(pl.program_id(0) == 0)
    def _():
        e2s_ref[:] = jnp.zeros_like(e2s_ref)

    e2s_ref[:] += jnp.sum(e2, axis=0, keepdims=True)


def _att_edge(ehv, dxv, sxv, uh_t, params):
    (W1, b1), (W2, b2) = params['att']['e']
    big = pl.BlockSpec((BLK, 128), lambda i: (i, 0))
    w = pl.BlockSpec((128, 128), lambda i: (0, 0))
    s = pl.BlockSpec((1, 128), lambda i: (0, 0))
    return pl.pallas_call(
        _att_edge_body,
        grid=(EV // BLK,),
        in_specs=[big, big, big, s, w, w, w, s, s],
        out_specs=[big, big, s],
        out_shape=[jax.ShapeDtypeStruct((EV, 128), _f32)] * 2
        + [jax.ShapeDtypeStruct((1, 128), _f32)],
    )(ehv, dxv, sxv, uh_t, _kron8(W1[32:48]), _kron8(W1[0:16]), _kron8(W2),
      _tile8(b1), _tile8(b2))


def _att_node_body(p0, p1, c0, c1, xh, uh_t, e2s, xp0r, k0n, k1n, kun, b1n,
                   k2n, b2n, w1u, b1u, w2u, b2u, w1g, b1g, w2g, b2g,
                   w1d, b1d, w2d, b2d, kxr, f, ft,
                   x2_ref, xpn_ref, u2_ref, dec_ref):
    seg = p0[:] + p1[:]
    agg = seg / jnp.maximum(c0[:] + c1[:], 1.0)
    xh_ = xh[:]
    h1 = jnp.maximum(_dot(xh_, k0n[:]) + _dot(agg, k1n[:])
                     + _dot(uh_t[:], kun[:]) + b1n[:], 0.0)
    x_a = _dot(h1, k2n[:]) + b2n[:]
    xa = _dot(jnp.sum(x_a, axis=0, keepdims=True), f[:]) * (1.0 / N_NODES)
    ea = _dot(jnp.sum(seg, axis=0, keepdims=True), f[:]) * (1.0 / N_EDGES)
    uh16 = uh_t[:, :H]
    ucat = jnp.concatenate([uh16, xa, ea], axis=1)
    hu = jnp.maximum(_dot(ucat, w1u[:]) + b1u[:, :H], 0.0)
    u_a = _dot(hu, w2u[:]) + b2u[:, :H]
    x2 = x_a * xh_
    x2_ref[:] = x2
    xpn_ref[:] = xp0r[:] + _dot(x2, kxr[:])
    u2 = u_a * uh16
    xa2 = _dot(jnp.sum(x2, axis=0, keepdims=True), f[:]) * (1.0 / N_NODES)
    ea2 = _dot(e2s[:], f[:]) * (1.0 / N_EDGES)
    gcat = jnp.concatenate([u2, xa2, ea2], axis=1)
    hg = jnp.maximum(_dot(gcat, w1g[:]) + b1g[:, :H], 0.0)
    u2p = _dot(hg, w2g[:]) + b2g[:, :H]
    u2_ref[:] = _dot(u2p, ft[:])
    hd = jnp.maximum(_dot(u2p, w1d[:]) + b1d[:, :H], 0.0)
    dec_ref[:] = _dot(_dot(hd, w2d[:]) + b2d[:, :H], ft[:])


def _att_node(parts, cnt, xhv, uh_t, e2s, xp0rv, kxrec, params):
    (W1n, b1n), (W2n, b2n) = params['att']['x']
    (W1u, b1u), (W2u, b2u) = params['att']['u']
    (W1g, b1g), (W2g, b2g) = params['agg']
    (W1d, b1d), (W2d, b2d) = params['dec']
    W2dp = jnp.zeros((H, H), _f32).at[:, :2].set(W2d)
    b2dp = jnp.zeros((H,), _f32).at[:2].set(b2d)
    f = jnp.tile(jnp.eye(H, dtype=_f32), (8, 1))
    ft = f.T
    return pl.pallas_call(
        _att_node_body,
        out_shape=[jax.ShapeDtypeStruct((NV, 128), _f32),
                   jax.ShapeDtypeStruct((NV, 128), _f32),
                   jax.ShapeDtypeStruct((1, 128), _f32),
                   jax.ShapeDtypeStruct((1, 128), _f32)],
    )(parts[0], parts[1], cnt[0], cnt[1], xhv, uh_t, e2s, xp0rv,
      _kron8(W1n[0:16]), _kron8(W1n[16:32]), _kron8(W1n[32:48]),
      _tile8(b1n), _kron8(W2n), _tile8(b2n), W1u, _tile8(b1u), W2u,
      _tile8(b2u), W1g, _tile8(b1g), W2g, _tile8(b2g), W1d, _tile8(b1d),
      W2dp, _tile8(b2dp), _kron8(kxrec), f, ft)


_NC, _NS = 2, 16
_NW = _NC * _NS
_EPW = N_EDGES // _NW
_GC = 2000
_NCH = _EPW // _GC
_NWB = 10
_NPT = N_NODES // _NWB

_sc_mesh = plsc.VectorSubcoreMesh(core_axis_name="c", subcore_axis_name="s")


_GCG = 1000
_NCHG = _EPW // _GCG


@functools.partial(
    pl.kernel,
    out_type=jax.ShapeDtypeStruct((2 * N_EDGES, H), _f32),
    mesh=_sc_mesh,
    compiler_params=pltpu.CompilerParams(use_tc_tiling_on_sc=False),
    scratch_types=[pltpu.VMEM((_EPW,), jnp.int32),
                   pltpu.VMEM((_EPW,), jnp.int32),
                   pltpu.VMEM((_GCG, H), _f32),
                   pltpu.VMEM((_GCG, H), _f32),
                   pltpu.VMEM((_GCG, H), _f32),
                   pltpu.VMEM((_GCG, H), _f32),
                   pltpu.SemaphoreType.DMA,
                   pltpu.SemaphoreType.DMA],
)
def _sc_gather2(xp_hbm, src_hbm, dst_hbm, out_hbm, idx_s, idx_d,
                rs0, rd0, rs1, rd1, sem0, sem1):
    wid = lax.axis_index("s") * _NC + lax.axis_index("c")
    base0 = pl.multiple_of(wid * _EPW, 8)
    pltpu.sync_copy(src_hbm.at[pl.ds(base0, _EPW)], idx_s)
    pltpu.sync_copy(dst_hbm.at[pl.ds(base0, _EPW)], idx_d)
    bufs = [(rs0, rd0, sem0), (rs1, rd1, sem1)]

    def issue(c):
        rs, rd, sem = bufs[c % 2]
        off = pl.multiple_of(c * _GCG, 8)
        h1 = pltpu.async_copy(xp_hbm.at[idx_s.at[pl.ds(off, _GCG)]], rs, sem)
        h2 = pltpu.async_copy(xp_hbm.at[idx_d.at[pl.ds(off, _GCG)]], rd, sem)
        return h1, h2

    def drain(c, hs):
        rs, rd, _ = bufs[c % 2]
        hs[0].wait()
        hs[1].wait()
        off = pl.multiple_of(c * _GCG, 8)
        pltpu.sync_copy(rd, out_hbm.at[pl.ds(base0 + off, _GCG)])
        pltpu.sync_copy(rs, out_hbm.at[pl.ds(
            pl.multiple_of(N_EDGES + base0 + off, 8), _GCG)])

    hs = issue(0)
    for c in range(_NCHG):
        nxt = issue(c + 1) if c + 1 < _NCHG else None
        drain(c, hs)
        hs = nxt


def _scatter_body(e_hbm, dst_hbm, zeros_hbm, ones_hbm, out_hbm, idx0, rows0,
                  idx1, rows1, sem0, sem1, ones_v, accum, accum_c, gc,
                  with_counts):
    cid = lax.axis_index("c")
    sid = lax.axis_index("s")
    wid = sid * _NC + cid
    off = pl.multiple_of(sid * _NPT, 8)
    nch = _EPW // gc

    @pl.when(sid < _NWB)
    def _():
        pltpu.sync_copy(zeros_hbm.at[pl.ds(0, _NPT)],
                        accum.at[pl.ds(off, _NPT)])
        if with_counts:
            pltpu.sync_copy(zeros_hbm.at[pl.ds(0, _NPT)],
                            accum_c.at[pl.ds(off, _NPT)])

    if with_counts:
        pltpu.sync_copy(ones_hbm, ones_v)
    plsc.subcore_barrier()
    base0 = pl.multiple_of(wid * _EPW, 8)
    bufs = [(idx0, rows0, sem0), (idx1, rows1, sem1)]

    def issue(c):
        idx_v, rows_v, sem = bufs[c % 2]
        base = pl.multiple_of(base0 + c * gc, 8)
        h1 = pltpu.async_copy(dst_hbm.at[pl.ds(base, gc)], idx_v, sem)
        h2 = pltpu.async_copy(e_hbm.at[pl.ds(base, gc)], rows_v, sem)
        return h1, h2

    def drain(c, hs):
        idx_v, rows_v, _ = bufs[c % 2]
        hs[0].wait()
        hs[1].wait()
        pltpu.sync_copy(rows_v, accum.at[idx_v], add=True)
        if with_counts:
            pltpu.sync_copy(ones_v, accum_c.at[idx_v], add=True)

    hs = issue(0)
    for c in range(nch):
        nxt = issue(c + 1) if c + 1 < nch else None
        drain(c, hs)
        hs = nxt

    plsc.subcore_barrier()

    @pl.when(sid < _NWB)
    def _():
        dst_off = pl.multiple_of(cid * N_NODES + sid * _NPT, 8)
        pltpu.sync_copy(accum.at[pl.ds(off, _NPT)],
                        out_hbm.at[pl.ds(dst_off, _NPT)])
        if with_counts:
            cnt_off = pl.multiple_of(
                2 * N_NODES + cid * N_NODES + sid * _NPT, 8)
            pltpu.sync_copy(accum_c.at[pl.ds(off, _NPT)],
                            out_hbm.at[pl.ds(cnt_off, _NPT)])


@functools.partial(
    pl.kernel,
    out_type=jax.ShapeDtypeStruct((2 * N_NODES, H), _f32),
    mesh=_sc_mesh,
    compiler_params=pltpu.CompilerParams(use_tc_tiling_on_sc=False),
    scratch_types=[pltpu.VMEM((_GC,), jnp.int32),
                   pltpu.VMEM((_GC, H), _f32),
                   pltpu.VMEM((_GC,), jnp.int32),
                   pltpu.VMEM((_GC, H), _f32),
                   pltpu.SemaphoreType.DMA,
                   pltpu.SemaphoreType.DMA,
                   pltpu.VMEM_SHARED((N_NODES, H), _f32)],
)
def _sc_scatter(e_hbm, dst_hbm, zeros_hbm, out_hbm, idx0, rows0, idx1,
                rows1, sem0, sem1, accum):
    _scatter_body(e_hbm, dst_hbm, zeros_hbm, None, out_hbm, idx0, rows0,
                  idx1, rows1, sem0, sem1, None, accum, None, _GC, False)


_GCC = 1000


@functools.partial(
    pl.kernel,
    out_type=jax.ShapeDtypeStruct((4 * N_NODES, H), _f32),
    mesh=_sc_mesh,
    compiler_params=pltpu.CompilerParams(use_tc_tiling_on_sc=False),
    scratch_types=[pltpu.VMEM((_GCC,), jnp.int32),
                   pltpu.VMEM((_GCC, H), _f32),
                   pltpu.VMEM((_GCC,), jnp.int32),
                   pltpu.VMEM((_GCC, H), _f32),
                   pltpu.SemaphoreType.DMA,
                   pltpu.SemaphoreType.DMA,
                   pltpu.VMEM((_GCC, H), _f32),
                   pltpu.VMEM_SHARED((N_NODES, H), _f32),
                   pltpu.VMEM_SHARED((N_NODES, H), _f32)],
)
def _sc_scatter_counts(e_hbm, dst_hbm, zeros_hbm, ones_hbm, out_hbm, idx0,
                       rows0, idx1, rows1, sem0, sem1, ones_v, accum,
                       accum_c):
    _scatter_body(e_hbm, dst_hbm, zeros_hbm, ones_hbm, out_hbm, idx0, rows0,
                  idx1, rows1, sem0, sem1, ones_v, accum, accum_c, _GCC,
                  True)


def _gather_diff(xp16, src, dst):
    gv = jnp.reshape(_sc_gather2(xp16, src, dst), (2, EV, 128))
    return gv[0], gv[1]


def _scatter_parts(e16, dst):
    zeros = jnp.zeros((_NPT, H), _f32)
    return jnp.reshape(_sc_scatter(e16, dst, zeros), (2, NV, 128))


def _scatter_parts_counts(e16, dst):
    zeros = jnp.zeros((_NPT, H), _f32)
    ones = jnp.ones((_GCC, H), _f32)
    return jnp.reshape(_sc_scatter_counts(e16, dst, zeros, ones),
                       (4, NV, 128))


def kernel(x1, edge_index1, e1, u1, batch1, x2, edge_index2, e2, u2, batch2,
           params):
    del batch1, batch2
    W1r_e = params['rec']['e'][0][0]
    W1a_e = params['att']['e'][0][0]
    kx_rec = W1r_e[160:176]
    kx_att = W1a_e[16:32]
    zero_nv = jnp.zeros((NV, 128), _f32)

    def prep(x, e, ei, u):
        src = ei[0]
        dst = ei[1]
        ev = jnp.reshape(e, (EV, 128))
        eh0v, e0v = _prep_edges(ev, params)
        xh0, xp0, xp1, xn0, uh0_t = _prep_nodes(x, _tile8(u), params)
        return dict(src=src, dst=dst, e0v=e0v, cnt=None,
                    xp0v=jnp.reshape(xp0, (NV, 128)),
                    xn0v=jnp.reshape(xn0, (NV, 128)),
                    ehv=eh0v, xhv=jnp.reshape(xh0, (NV, 128)),
                    xpv=jnp.reshape(xp1, (NV, 128)), uh_t=uh0_t)

    g1 = prep(x1, e1, edge_index1, u1)
    g2 = prep(x2, e2, edge_index2, u2)
    u1_t = _tile8(u1)
    u2_t = _tile8(u2)

    def processing(g, u_t, shared_t, first):
        xhv, ehv, xpv, uh_t = g['xhv'], g['ehv'], g['xpv'], g['uh_t']
        gn = dict(g)
        for inner in range(2):
            dxv, sxv = _gather_diff(jnp.reshape(xpv, (N_NODES, H)),
                                    g['src'], g['dst'])
            env = _rec_edge(g['e0v'], ehv, dxv, sxv, u_t, shared_t, params)
            if first and inner == 0:
                parts4 = _scatter_parts_counts(jnp.reshape(env, (N_EDGES, H)),
                                               g['dst'])
                parts = parts4[:2]
                gn['cnt'] = parts4[2:]
            else:
                parts = _scatter_parts(jnp.reshape(env, (N_EDGES, H)),
                                       g['dst'])
            kxn = kx_rec if inner == 0 else kx_att
            xp0n = g['xp0v'] if inner == 0 else zero_nv
            xhv, xpv, uh_t = _rec_node(parts, gn['cnt'], g['xn0v'], xhv,
                                       u_t, shared_t, xp0n, kxn, params)
            ehv = env
        dxv, sxv = _gather_diff(jnp.reshape(xpv, (N_NODES, H)),
                                g['src'], g['dst'])
        eav, e2v, e2s = _att_edge(ehv, dxv, sxv, uh_t, params)
        parts = _scatter_parts(jnp.reshape(eav, (N_EDGES, H)), g['dst'])
        x2v, xpnv, u2_t, dec_t = _att_node(parts, gn['cnt'], xhv, uh_t, e2s,
                                           g['xp0v'], kx_rec, params)
        gn['xhv'], gn['ehv'], gn['xpv'], gn['uh_t'] = x2v, e2v, xpnv, u2_t
        return gn, dec_t

    outs = []
    for p in range(2):
        g1, _dec1 = processing(g1, u1_t, g2['uh_t'], p == 0)
        g2, dec2 = processing(g2, u2_t, g1['uh_t'], p == 0)
        outs.append(dec2[:, :2])
    return jnp.stack(outs)


# speedup vs baseline: 3.7356x; 3.7356x over previous
import functools

import jax
import jax.numpy as jnp
from jax import lax
from jax.experimental import pallas as pl
from jax.experimental.pallas import tpu as pltpu
from jax.experimental.pallas import tpu_sc as plsc

N_NODES = 10000
N_EDGES = 320000
EV = N_EDGES // 8
NV = N_NODES // 8
BLK = 5000
H = 16

_f32 = jnp.float32


def _kron8(W):
    return jnp.kron(jnp.eye(8, dtype=_f32), W.astype(_f32))


def _tile8(v):
    return jnp.tile(jnp.reshape(v, (1, H)), (1, 8))


def _dot(a, b):
    return jnp.dot(a, b, preferred_element_type=_f32)


def _prep_edges_body(e_ref, ke1, be1, ke2, be2, k0, b1e, eh0_ref, e0_ref):
    e = e_ref[:]
    h = jnp.maximum(_dot(e, ke1[:]) + be1[:], 0.0)
    eh0_ref[:] = _dot(h, ke2[:]) + be2[:]
    e0_ref[:] = _dot(e, k0[:]) + b1e[:]


def _prep_edges(ev, params):
    (W1, b1), (W2, b2) = params['enc']['e']
    W1r, b1r = params['rec']['e'][0]
    big = pl.BlockSpec((BLK, 128), lambda i: (i, 0))
    w = pl.BlockSpec((128, 128), lambda i: (0, 0))
    s = pl.BlockSpec((1, 128), lambda i: (0, 0))
    return pl.pallas_call(
        _prep_edges_body,
        grid=(EV // BLK,),
        in_specs=[big, w, s, w, s, w, s],
        out_specs=[big, big],
        out_shape=[jax.ShapeDtypeStruct((EV, 128), _f32)] * 2,
    )(ev, _kron8(W1), _tile8(b1), _kron8(W2), _tile8(b2),
      _kron8(W1r[0:16]), _tile8(b1r))


def _prep_nodes_body(x_ref, u_ref, w1x, b1x, w2x, b2x, wxp, wxn,
                     wxp1, w1u, b1u, w2u, b2u, ft,
                     xh0_ref, xp0_ref, xp1_ref, xn0_ref, uh0_ref):
    x = x_ref[:]
    h = jnp.maximum(_dot(x, w1x[:]) + b1x[:], 0.0)
    xh0 = _dot(h, w2x[:]) + b2x[:]
    xh0_ref[:] = xh0
    xp0 = _dot(x, wxp[:])
    xp0_ref[:] = xp0
    xp1_ref[:] = xp0 + _dot(xh0, wxp1[:])
    xn0_ref[:] = _dot(x, wxn[:])
    u16 = u_ref[:, :H]
    hu = jnp.maximum(_dot(u16, w1u[:]) + b1u[:], 0.0)
    uh0_ref[:] = _dot(_dot(hu, w2u[:]) + b2u[:], ft[:])


def _prep_nodes(x, u_t, params):
    (W1x, b1x), (W2x, b2x) = params['enc']['x']
    (W1u, b1u), (W2u, b2u) = params['enc']['u']
    W1r, _ = params['rec']['e'][0]
    W1n, _ = params['rec']['x'][0]
    ft = jnp.tile(jnp.eye(H, dtype=_f32), (1, 8))
    outs = pl.pallas_call(
        _prep_nodes_body,
        out_shape=[jax.ShapeDtypeStruct((N_NODES, H), _f32)] * 4
        + [jax.ShapeDtypeStruct((1, 128), _f32)],
    )(x, u_t, W1x, jnp.reshape(b1x, (1, H)), W2x,
      jnp.reshape(b2x, (1, H)), W1r[32:160], W1n[0:128], W1r[160:176],
      W1u, jnp.reshape(b1u, (1, H)), W2u, jnp.reshape(b2u, (1, H)), ft)
    return outs


def _edge_mlp_body(e0_ref, eh_ref, dx_ref, sx_ref, u_t, s_t, ku, ks, k16,
                   k2, b2, out_ref):
    ut = _dot(u_t[:], ku[:]) + _dot(s_t[:], ks[:])
    h1 = jnp.maximum(e0_ref[:] + _dot(eh_ref[:], k16[:])
                     + (dx_ref[:] - sx_ref[:]) + ut, 0.0)
    out_ref[:] = _dot(h1, k2[:]) + b2[:]


def _rec_edge(e0v, ehv, dxv, sxv, u_t, s_t, params):
    (W1, _), (W2, b2) = params['rec']['e']
    big = pl.BlockSpec((BLK, 128), lambda i: (i, 0))
    w = pl.BlockSpec((128, 128), lambda i: (0, 0))
    s = pl.BlockSpec((1, 128), lambda i: (0, 0))
    return pl.pallas_call(
        _edge_mlp_body,
        grid=(EV // BLK,),
        in_specs=[big, big, big, big, s, s, w, w, w, w, s],
        out_specs=big,
        out_shape=jax.ShapeDtypeStruct((EV, 128), _f32),
    )(e0v, ehv, dxv, sxv, u_t, s_t, _kron8(W1[176:192]),
      _kron8(W1[192:208]), _kron8(W1[16:32]), _kron8(W2), _tile8(b2))


def _rec_node_body(p0, p1, c0, c1, xn0, xh, u_t, s_t, xp0n, kb, kc, kun, ksn,
                   b1n, k2n, b2n, w1u, b1u, w2u, b2u, kxn, f, ft,
                   xnew_ref, xpn_ref, unew_ref):
    seg = p0[:] + p1[:]
    agg = seg / jnp.maximum(c0[:] + c1[:], 1.0)
    un = _dot(u_t[:], kun[:]) + _dot(s_t[:], ksn[:])
    h1 = jnp.maximum(xn0[:] + _dot(xh[:], kb[:]) + _dot(agg, kc[:]) + un
                     + b1n[:], 0.0)
    xnew = _dot(h1, k2n[:]) + b2n[:]
    xnew_ref[:] = xnew
    xpn_ref[:] = xp0n[:] + _dot(xnew, kxn[:])
    xa = _dot(jnp.sum(xnew, axis=0, keepdims=True), f[:]) * (1.0 / N_NODES)
    ea = _dot(jnp.sum(seg, axis=0, keepdims=True), f[:]) * (1.0 / N_EDGES)
    ucat = jnp.concatenate([u_t[:, :H], s_t[:, :H], xa, ea], axis=1)
    hu = jnp.maximum(_dot(ucat, w1u[:]) + b1u[:, :H], 0.0)
    unew_ref[:] = _dot(_dot(hu, w2u[:]) + b2u[:, :H], ft[:])


def _rec_node(parts, cnt, xn0v, xhv, u_t, s_t, xp0nextv, kxnext, params):
    (W1n, b1n), (W2n, b2n) = params['rec']['x']
    (W1u, b1u), (W2u, b2u) = params['rec']['u']
    f = jnp.tile(jnp.eye(H, dtype=_f32), (8, 1))
    ft = f.T
    return pl.pallas_call(
        _rec_node_body,
        out_shape=[jax.ShapeDtypeStruct((NV, 128), _f32),
                   jax.ShapeDtypeStruct((NV, 128), _f32),
                   jax.ShapeDtypeStruct((1, 128), _f32)],
    )(parts[0], parts[1], cnt[0], cnt[1], xn0v, xhv, u_t, s_t, xp0nextv,
      _kron8(W1n[128:144]), _kron8(W1n[144:160]), _kron8(W1n[160:176]),
      _kron8(W1n[176:192]), _tile8(b1n), _kron8(W2n), _tile8(b2n),
      W1u, _tile8(b1u), W2u, _tile8(b2u), _kron8(kxnext), f, ft)


def _att_edge_body(eh_ref, dx_ref, sx_ref, uh_t, ku, k0, k2, b1, b2,
                   ea_ref, e2_ref, e2s_ref):
    ut = _dot(uh_t[:], ku[:]) + b1[:]
    eh = eh_ref[:]
    h1 = jnp.maximum(_dot(eh, k0[:]) + (dx_ref[:] - sx_ref[:]) + ut, 0.0)
    ea = _dot(h1, k2[:]) + b2[:]
    ea_ref[:] = ea
    e2 = ea * eh
    e2_ref[:] = e2

    @pl.when(pl.program_id(0) == 0)
    def _():
        e2s_ref[:] = jnp.zeros_like(e2s_ref)

    e2s_ref[:] += jnp.sum(e2, axis=0, keepdims=True)


def _att_edge(ehv, dxv, sxv, uh_t, params):
    (W1, b1), (W2, b2) = params['att']['e']
    big = pl.BlockSpec((BLK, 128), lambda i: (i, 0))
    w = pl.BlockSpec((128, 128), lambda i: (0, 0))
    s = pl.BlockSpec((1, 128), lambda i: (0, 0))
    return pl.pallas_call(
        _att_edge_body,
        grid=(EV // BLK,),
        in_specs=[big, big, big, s, w, w, w, s, s],
        out_specs=[big, big, s],
        out_shape=[jax.ShapeDtypeStruct((EV, 128), _f32)] * 2
        + [jax.ShapeDtypeStruct((1, 128), _f32)],
    )(ehv, dxv, sxv, uh_t, _kron8(W1[32:48]), _kron8(W1[0:16]), _kron8(W2),
      _tile8(b1), _tile8(b2))


def _att_node_body(p0, p1, c0, c1, xh, uh_t, e2s, xp0r, k0n, k1n, kun, b1n,
                   k2n, b2n, w1u, b1u, w2u, b2u, w1g, b1g, w2g, b2g,
                   w1d, b1d, w2d, b2d, kxr, f, ft,
                   x2_ref, xpn_ref, u2_ref, dec_ref):
    seg = p0[:] + p1[:]
    agg = seg / jnp.maximum(c0[:] + c1[:], 1.0)
    xh_ = xh[:]
    h1 = jnp.maximum(_dot(xh_, k0n[:]) + _dot(agg, k1n[:])
                     + _dot(uh_t[:], kun[:]) + b1n[:], 0.0)
    x_a = _dot(h1, k2n[:]) + b2n[:]
    xa = _dot(jnp.sum(x_a, axis=0, keepdims=True), f[:]) * (1.0 / N_NODES)
    ea = _dot(jnp.sum(seg, axis=0, keepdims=True), f[:]) * (1.0 / N_EDGES)
    uh16 = uh_t[:, :H]
    ucat = jnp.concatenate([uh16, xa, ea], axis=1)
    hu = jnp.maximum(_dot(ucat, w1u[:]) + b1u[:, :H], 0.0)
    u_a = _dot(hu, w2u[:]) + b2u[:, :H]
    x2 = x_a * xh_
    x2_ref[:] = x2
    xpn_ref[:] = xp0r[:] + _dot(x2, kxr[:])
    u2 = u_a * uh16
    xa2 = _dot(jnp.sum(x2, axis=0, keepdims=True), f[:]) * (1.0 / N_NODES)
    ea2 = _dot(e2s[:], f[:]) * (1.0 / N_EDGES)
    gcat = jnp.concatenate([u2, xa2, ea2], axis=1)
    hg = jnp.maximum(_dot(gcat, w1g[:]) + b1g[:, :H], 0.0)
    u2p = _dot(hg, w2g[:]) + b2g[:, :H]
    u2_ref[:] = _dot(u2p, ft[:])
    hd = jnp.maximum(_dot(u2p, w1d[:]) + b1d[:, :H], 0.0)
    dec_ref[:] = _dot(_dot(hd, w2d[:]) + b2d[:, :H], ft[:])


def _att_node(parts, cnt, xhv, uh_t, e2s, xp0rv, kxrec, params):
    (W1n, b1n), (W2n, b2n) = params['att']['x']
    (W1u, b1u), (W2u, b2u) = params['att']['u']
    (W1g, b1g), (W2g, b2g) = params['agg']
    (W1d, b1d), (W2d, b2d) = params['dec']
    W2dp = jnp.zeros((H, H), _f32).at[:, :2].set(W2d)
    b2dp = jnp.zeros((H,), _f32).at[:2].set(b2d)
    f = jnp.tile(jnp.eye(H, dtype=_f32), (8, 1))
    ft = f.T
    return pl.pallas_call(
        _att_node_body,
        out_shape=[jax.ShapeDtypeStruct((NV, 128), _f32),
                   jax.ShapeDtypeStruct((NV, 128), _f32),
                   jax.ShapeDtypeStruct((1, 128), _f32),
                   jax.ShapeDtypeStruct((1, 128), _f32)],
    )(parts[0], parts[1], cnt[0], cnt[1], xhv, uh_t, e2s, xp0rv,
      _kron8(W1n[0:16]), _kron8(W1n[16:32]), _kron8(W1n[32:48]),
      _tile8(b1n), _kron8(W2n), _tile8(b2n), W1u, _tile8(b1u), W2u,
      _tile8(b2u), W1g, _tile8(b1g), W2g, _tile8(b2g), W1d, _tile8(b1d),
      W2dp, _tile8(b2dp), _kron8(kxrec), f, ft)


_NC, _NS = 2, 16
_NW = _NC * _NS
_EPW = N_EDGES // _NW
_GC = 2000
_NCH = _EPW // _GC
_NWB = 10
_NPT = N_NODES // _NWB

_sc_mesh = plsc.VectorSubcoreMesh(core_axis_name="c", subcore_axis_name="s")


_GCG = 1000
_NCHG = _EPW // _GCG


@functools.partial(
    pl.kernel,
    out_type=[jax.ShapeDtypeStruct((N_EDGES, H), _f32),
              jax.ShapeDtypeStruct((N_EDGES, H), _f32)],
    mesh=_sc_mesh,
    compiler_params=pltpu.CompilerParams(use_tc_tiling_on_sc=False),
    scratch_types=[pltpu.VMEM((_EPW,), jnp.int32),
                   pltpu.VMEM((_EPW,), jnp.int32),
                   pltpu.VMEM((_GCG, H), _f32),
                   pltpu.VMEM((_GCG, H), _f32),
                   pltpu.VMEM((_GCG, H), _f32),
                   pltpu.VMEM((_GCG, H), _f32),
                   pltpu.SemaphoreType.DMA,
                   pltpu.SemaphoreType.DMA],
)
def _sc_gather2(xp_hbm, src_hbm, dst_hbm, outd_hbm, outs_hbm, idx_s, idx_d,
                rs0, rd0, rs1, rd1, sem0, sem1):
    wid = lax.axis_index("s") * _NC + lax.axis_index("c")
    base0 = pl.multiple_of(wid * _EPW, 8)
    pltpu.sync_copy(src_hbm.at[pl.ds(base0, _EPW)], idx_s)
    pltpu.sync_copy(dst_hbm.at[pl.ds(base0, _EPW)], idx_d)
    bufs = [(rs0, rd0, sem0), (rs1, rd1, sem1)]

    def issue(c):
        rs, rd, sem = bufs[c % 2]
        off = pl.multiple_of(c * _GCG, 8)
        h1 = pltpu.async_copy(xp_hbm.at[idx_s.at[pl.ds(off, _GCG)]], rs, sem)
        h2 = pltpu.async_copy(xp_hbm.at[idx_d.at[pl.ds(off, _GCG)]], rd, sem)
        return h1, h2

    def drain(c, hs):
        rs, rd, _ = bufs[c % 2]
        hs[0].wait()
        hs[1].wait()
        off = pl.multiple_of(c * _GCG, 8)
        pltpu.sync_copy(rd, outd_hbm.at[pl.ds(base0 + off, _GCG)])
        pltpu.sync_copy(rs, outs_hbm.at[pl.ds(base0 + off, _GCG)])

    hs = issue(0)
    for c in range(_NCHG):
        nxt = issue(c + 1) if c + 1 < _NCHG else None
        drain(c, hs)
        hs = nxt


def _scatter_body(e_hbm, dst_hbm, zeros_hbm, ones_hbm, out_hbm, idx0, rows0,
                  idx1, rows1, sem0, sem1, ones_v, accum, accum_c, gc,
                  with_counts):
    cid = lax.axis_index("c")
    sid = lax.axis_index("s")
    wid = sid * _NC + cid
    off = pl.multiple_of(sid * _NPT, 8)
    nch = _EPW // gc

    @pl.when(sid < _NWB)
    def _():
        pltpu.sync_copy(zeros_hbm.at[pl.ds(0, _NPT)],
                        accum.at[pl.ds(off, _NPT)])
        if with_counts:
            pltpu.sync_copy(zeros_hbm.at[pl.ds(0, _NPT)],
                            accum_c.at[pl.ds(off, _NPT)])

    if with_counts:
        pltpu.sync_copy(ones_hbm, ones_v)
    plsc.subcore_barrier()
    base0 = pl.multiple_of(wid * _EPW, 8)
    bufs = [(idx0, rows0, sem0), (idx1, rows1, sem1)]

    def issue(c):
        idx_v, rows_v, sem = bufs[c % 2]
        base = pl.multiple_of(base0 + c * gc, 8)
        h1 = pltpu.async_copy(dst_hbm.at[pl.ds(base, gc)], idx_v, sem)
        h2 = pltpu.async_copy(e_hbm.at[pl.ds(base, gc)], rows_v, sem)
        return h1, h2

    def drain(c, hs):
        idx_v, rows_v, _ = bufs[c % 2]
        hs[0].wait()
        hs[1].wait()
        pltpu.sync_copy(rows_v, accum.at[idx_v], add=True)
        if with_counts:
            pltpu.sync_copy(ones_v, accum_c.at[idx_v], add=True)

    hs = issue(0)
    for c in range(nch):
        nxt = issue(c + 1) if c + 1 < nch else None
        drain(c, hs)
        hs = nxt

    plsc.subcore_barrier()

    @pl.when(sid < _NWB)
    def _():
        dst_off = pl.multiple_of(cid * N_NODES + sid * _NPT, 8)
        pltpu.sync_copy(accum.at[pl.ds(off, _NPT)],
                        out_hbm.at[pl.ds(dst_off, _NPT)])
        if with_counts:
            cnt_off = pl.multiple_of(
                2 * N_NODES + cid * N_NODES + sid * _NPT, 8)
            pltpu.sync_copy(accum_c.at[pl.ds(off, _NPT)],
                            out_hbm.at[pl.ds(cnt_off, _NPT)])


@functools.partial(
    pl.kernel,
    out_type=jax.ShapeDtypeStruct((2 * N_NODES, H), _f32),
    mesh=_sc_mesh,
    compiler_params=pltpu.CompilerParams(use_tc_tiling_on_sc=False),
    scratch_types=[pltpu.VMEM((_GC,), jnp.int32),
                   pltpu.VMEM((_GC, H), _f32),
                   pltpu.VMEM((_GC,), jnp.int32),
                   pltpu.VMEM((_GC, H), _f32),
                   pltpu.SemaphoreType.DMA,
                   pltpu.SemaphoreType.DMA,
                   pltpu.VMEM_SHARED((N_NODES, H), _f32)],
)
def _sc_scatter(e_hbm, dst_hbm, zeros_hbm, out_hbm, idx0, rows0, idx1,
                rows1, sem0, sem1, accum):
    _scatter_body(e_hbm, dst_hbm, zeros_hbm, None, out_hbm, idx0, rows0,
                  idx1, rows1, sem0, sem1, None, accum, None, _GC, False)


_GCC = 1000


@functools.partial(
    pl.kernel,
    out_type=jax.ShapeDtypeStruct((4 * N_NODES, H), _f32),
    mesh=_sc_mesh,
    compiler_params=pltpu.CompilerParams(use_tc_tiling_on_sc=False),
    scratch_types=[pltpu.VMEM((_GCC,), jnp.int32),
                   pltpu.VMEM((_GCC, H), _f32),
                   pltpu.VMEM((_GCC,), jnp.int32),
                   pltpu.VMEM((_GCC, H), _f32),
                   pltpu.SemaphoreType.DMA,
                   pltpu.SemaphoreType.DMA,
                   pltpu.VMEM((_GCC, H), _f32),
                   pltpu.VMEM_SHARED((N_NODES, H), _f32),
                   pltpu.VMEM_SHARED((N_NODES, H), _f32)],
)
def _sc_scatter_counts(e_hbm, dst_hbm, zeros_hbm, ones_hbm, out_hbm, idx0,
                       rows0, idx1, rows1, sem0, sem1, ones_v, accum,
                       accum_c):
    _scatter_body(e_hbm, dst_hbm, zeros_hbm, ones_hbm, out_hbm, idx0, rows0,
                  idx1, rows1, sem0, sem1, ones_v, accum, accum_c, _GCC,
                  True)


def _gather_diff(xp16, src, dst):
    gd, gs = _sc_gather2(xp16, src, dst)
    return jnp.reshape(gd, (EV, 128)), jnp.reshape(gs, (EV, 128))


def _scatter_parts(e16, dst):
    zeros = jnp.zeros((_NPT, H), _f32)
    return jnp.reshape(_sc_scatter(e16, dst, zeros), (2, NV, 128))


def _scatter_parts_counts(e16, dst):
    zeros = jnp.zeros((_NPT, H), _f32)
    ones = jnp.ones((_GCC, H), _f32)
    return jnp.reshape(_sc_scatter_counts(e16, dst, zeros, ones),
                       (4, NV, 128))


def kernel(x1, edge_index1, e1, u1, batch1, x2, edge_index2, e2, u2, batch2,
           params):
    del batch1, batch2
    W1r_e = params['rec']['e'][0][0]
    W1a_e = params['att']['e'][0][0]
    kx_rec = W1r_e[160:176]
    kx_att = W1a_e[16:32]
    zero_nv = jnp.zeros((NV, 128), _f32)

    def prep(x, e, ei, u):
        src = ei[0]
        dst = ei[1]
        ev = jnp.reshape(e, (EV, 128))
        eh0v, e0v = _prep_edges(ev, params)
        xh0, xp0, xp1, xn0, uh0_t = _prep_nodes(x, _tile8(u), params)
        return dict(src=src, dst=dst, e0v=e0v, cnt=None,
                    xp0v=jnp.reshape(xp0, (NV, 128)),
                    xn0v=jnp.reshape(xn0, (NV, 128)),
                    ehv=eh0v, xhv=jnp.reshape(xh0, (NV, 128)),
                    xpv=jnp.reshape(xp1, (NV, 128)), uh_t=uh0_t)

    g1 = prep(x1, e1, edge_index1, u1)
    g2 = prep(x2, e2, edge_index2, u2)
    u1_t = _tile8(u1)
    u2_t = _tile8(u2)

    def processing(g, u_t, shared_t, first):
        xhv, ehv, xpv, uh_t = g['xhv'], g['ehv'], g['xpv'], g['uh_t']
        gn = dict(g)
        for inner in range(2):
            dxv, sxv = _gather_diff(jnp.reshape(xpv, (N_NODES, H)),
                                    g['src'], g['dst'])
            env = _rec_edge(g['e0v'], ehv, dxv, sxv, u_t, shared_t, params)
            if first and inner == 0:
                parts4 = _scatter_parts_counts(jnp.reshape(env, (N_EDGES, H)),
                                               g['dst'])
                parts = parts4[:2]
                gn['cnt'] = parts4[2:]
            else:
                parts = _scatter_parts(jnp.reshape(env, (N_EDGES, H)),
                                       g['dst'])
            kxn = kx_rec if inner == 0 else kx_att
            xp0n = g['xp0v'] if inner == 0 else zero_nv
            xhv, xpv, uh_t = _rec_node(parts, gn['cnt'], g['xn0v'], xhv,
                                       u_t, shared_t, xp0n, kxn, params)
            ehv = env
        dxv, sxv = _gather_diff(jnp.reshape(xpv, (N_NODES, H)),
                                g['src'], g['dst'])
        eav, e2v, e2s = _att_edge(ehv, dxv, sxv, uh_t, params)
        parts = _scatter_parts(jnp.reshape(eav, (N_EDGES, H)), g['dst'])
        x2v, xpnv, u2_t, dec_t = _att_node(parts, gn['cnt'], xhv, uh_t, e2s,
                                           g['xp0v'], kx_rec, params)
        gn['xhv'], gn['ehv'], gn['xpv'], gn['uh_t'] = x2v, e2v, xpnv, u2_t
        return gn, dec_t

    outs = []
    for p in range(2):
        g1, _dec1 = processing(g1, u1_t, g2['uh_t'], p == 0)
        g2, dec2 = processing(g2, u2_t, g1['uh_t'], p == 0)
        outs.append(dec2[:, :2])
    return jnp.stack(outs)


# speedup vs baseline: 3.8988x; 1.0437x over previous
import functools

import jax
import jax.numpy as jnp
from jax import lax
from jax.experimental import pallas as pl
from jax.experimental.pallas import tpu as pltpu
from jax.experimental.pallas import tpu_sc as plsc

N_NODES = 10000
N_EDGES = 320000
EV = N_EDGES // 8
NV = N_NODES // 8
BLK = 5000
H = 16

_f32 = jnp.float32


def _kron8(W):
    return jnp.kron(jnp.eye(8, dtype=_f32), W.astype(_f32))


def _tile8(v):
    return jnp.tile(jnp.reshape(v, (1, H)), (1, 8))


def _dot(a, b):
    return jnp.dot(a, b, preferred_element_type=_f32)


def _prep_edges_body(e_ref, ke1, be1, ke2, be2, k0, b1e, eh0_ref, e0_ref):
    e = e_ref[:]
    h = jnp.maximum(_dot(e, ke1[:]) + be1[:], 0.0)
    eh0_ref[:] = _dot(h, ke2[:]) + be2[:]
    e0_ref[:] = _dot(e, k0[:]) + b1e[:]


def _prep_edges(ev, params):
    (W1, b1), (W2, b2) = params['enc']['e']
    W1r, b1r = params['rec']['e'][0]
    big = pl.BlockSpec((BLK, 128), lambda i: (i, 0))
    w = pl.BlockSpec((128, 128), lambda i: (0, 0))
    s = pl.BlockSpec((1, 128), lambda i: (0, 0))
    return pl.pallas_call(
        _prep_edges_body,
        grid=(EV // BLK,),
        in_specs=[big, w, s, w, s, w, s],
        out_specs=[big, big],
        out_shape=[jax.ShapeDtypeStruct((EV, 128), _f32)] * 2,
    )(ev, _kron8(W1), _tile8(b1), _kron8(W2), _tile8(b2),
      _kron8(W1r[0:16]), _tile8(b1r))


def _prep_nodes_body(x_ref, u_ref, w1x, b1x, w2x, b2x, wxp, wxn,
                     wxp1, w1u, b1u, w2u, b2u, ft,
                     xh0_ref, xp0_ref, xp1_ref, xn0_ref, uh0_ref):
    x = x_ref[:]
    h = jnp.maximum(_dot(x, w1x[:]) + b1x[:], 0.0)
    xh0 = _dot(h, w2x[:]) + b2x[:]
    xh0_ref[:] = xh0
    xp0 = _dot(x, wxp[:])
    xp0_ref[:] = xp0
    xp1_ref[:] = xp0 + _dot(xh0, wxp1[:])
    xn0_ref[:] = _dot(x, wxn[:])
    u16 = u_ref[:, :H]
    hu = jnp.maximum(_dot(u16, w1u[:]) + b1u[:], 0.0)
    uh0_ref[:] = _dot(_dot(hu, w2u[:]) + b2u[:], ft[:])


def _prep_nodes(x, u_t, params):
    (W1x, b1x), (W2x, b2x) = params['enc']['x']
    (W1u, b1u), (W2u, b2u) = params['enc']['u']
    W1r, _ = params['rec']['e'][0]
    W1n, _ = params['rec']['x'][0]
    ft = jnp.tile(jnp.eye(H, dtype=_f32), (1, 8))
    outs = pl.pallas_call(
        _prep_nodes_body,
        out_shape=[jax.ShapeDtypeStruct((N_NODES, H), _f32)] * 4
        + [jax.ShapeDtypeStruct((1, 128), _f32)],
    )(x, u_t, W1x, jnp.reshape(b1x, (1, H)), W2x,
      jnp.reshape(b2x, (1, H)), W1r[32:160], W1n[0:128], W1r[160:176],
      W1u, jnp.reshape(b1u, (1, H)), W2u, jnp.reshape(b2u, (1, H)), ft)
    return outs


def _edge_mlp_body(e0_ref, eh_ref, dx_ref, u_t, s_t, ku, ks, k16,
                   k2, b2, out_ref):
    ut = _dot(u_t[:], ku[:]) + _dot(s_t[:], ks[:])
    h1 = jnp.maximum(e0_ref[:] + _dot(eh_ref[:], k16[:]) + dx_ref[:] + ut,
                     0.0)
    out_ref[:] = _dot(h1, k2[:]) + b2[:]


def _rec_edge(e0v, ehv, dxv, u_t, s_t, params):
    (W1, _), (W2, b2) = params['rec']['e']
    big = pl.BlockSpec((BLK, 128), lambda i: (i, 0))
    w = pl.BlockSpec((128, 128), lambda i: (0, 0))
    s = pl.BlockSpec((1, 128), lambda i: (0, 0))
    return pl.pallas_call(
        _edge_mlp_body,
        grid=(EV // BLK,),
        in_specs=[big, big, big, s, s, w, w, w, w, s],
        out_specs=big,
        out_shape=jax.ShapeDtypeStruct((EV, 128), _f32),
    )(e0v, ehv, dxv, u_t, s_t, _kron8(W1[176:192]),
      _kron8(W1[192:208]), _kron8(W1[16:32]), _kron8(W2), _tile8(b2))


def _rec_node_body(p0, p1, c0, c1, xn0, xh, u_t, s_t, xp0n, kb, kc, kun, ksn,
                   b1n, k2n, b2n, w1u, b1u, w2u, b2u, kxn, f, ft,
                   xnew_ref, xpn_ref, unew_ref):
    seg = p0[:] + p1[:]
    agg = seg / jnp.maximum(c0[:] + c1[:], 1.0)
    un = _dot(u_t[:], kun[:]) + _dot(s_t[:], ksn[:])
    h1 = jnp.maximum(xn0[:] + _dot(xh[:], kb[:]) + _dot(agg, kc[:]) + un
                     + b1n[:], 0.0)
    xnew = _dot(h1, k2n[:]) + b2n[:]
    xnew_ref[:] = xnew
    xpn_ref[:] = xp0n[:] + _dot(xnew, kxn[:])
    xa = _dot(jnp.sum(xnew, axis=0, keepdims=True), f[:]) * (1.0 / N_NODES)
    ea = _dot(jnp.sum(seg, axis=0, keepdims=True), f[:]) * (1.0 / N_EDGES)
    ucat = jnp.concatenate([u_t[:, :H], s_t[:, :H], xa, ea], axis=1)
    hu = jnp.maximum(_dot(ucat, w1u[:]) + b1u[:, :H], 0.0)
    unew_ref[:] = _dot(_dot(hu, w2u[:]) + b2u[:, :H], ft[:])


def _rec_node(parts, cnt, xn0v, xhv, u_t, s_t, xp0nextv, kxnext, params):
    (W1n, b1n), (W2n, b2n) = params['rec']['x']
    (W1u, b1u), (W2u, b2u) = params['rec']['u']
    f = jnp.tile(jnp.eye(H, dtype=_f32), (8, 1))
    ft = f.T
    return pl.pallas_call(
        _rec_node_body,
        out_shape=[jax.ShapeDtypeStruct((NV, 128), _f32),
                   jax.ShapeDtypeStruct((NV, 128), _f32),
                   jax.ShapeDtypeStruct((1, 128), _f32)],
    )(parts[0], parts[1], cnt[0], cnt[1], xn0v, xhv, u_t, s_t, xp0nextv,
      _kron8(W1n[128:144]), _kron8(W1n[144:160]), _kron8(W1n[160:176]),
      _kron8(W1n[176:192]), _tile8(b1n), _kron8(W2n), _tile8(b2n),
      W1u, _tile8(b1u), W2u, _tile8(b2u), _kron8(kxnext), f, ft)


def _att_edge_body(eh_ref, dx_ref, uh_t, ku, k0, k2, b1, b2,
                   ea_ref, e2_ref, e2s_ref):
    ut = _dot(uh_t[:], ku[:]) + b1[:]
    eh = eh_ref[:]
    h1 = jnp.maximum(_dot(eh, k0[:]) + dx_ref[:] + ut, 0.0)
    ea = _dot(h1, k2[:]) + b2[:]
    ea_ref[:] = ea
    e2 = ea * eh
    e2_ref[:] = e2

    @pl.when(pl.program_id(0) == 0)
    def _():
        e2s_ref[:] = jnp.zeros_like(e2s_ref)

    e2s_ref[:] += jnp.sum(e2, axis=0, keepdims=True)


def _att_edge(ehv, dxv, uh_t, params):
    (W1, b1), (W2, b2) = params['att']['e']
    big = pl.BlockSpec((BLK, 128), lambda i: (i, 0))
    w = pl.BlockSpec((128, 128), lambda i: (0, 0))
    s = pl.BlockSpec((1, 128), lambda i: (0, 0))
    return pl.pallas_call(
        _att_edge_body,
        grid=(EV // BLK,),
        in_specs=[big, big, s, w, w, w, s, s],
        out_specs=[big, big, s],
        out_shape=[jax.ShapeDtypeStruct((EV, 128), _f32)] * 2
        + [jax.ShapeDtypeStruct((1, 128), _f32)],
    )(ehv, dxv, uh_t, _kron8(W1[32:48]), _kron8(W1[0:16]), _kron8(W2),
      _tile8(b1), _tile8(b2))


def _att_node_body(p0, p1, c0, c1, xh, uh_t, e2s, xp0r, k0n, k1n, kun, b1n,
                   k2n, b2n, w1u, b1u, w2u, b2u, w1g, b1g, w2g, b2g,
                   w1d, b1d, w2d, b2d, kxr, f, ft,
                   x2_ref, xpn_ref, u2_ref, dec_ref):
    seg = p0[:] + p1[:]
    agg = seg / jnp.maximum(c0[:] + c1[:], 1.0)
    xh_ = xh[:]
    h1 = jnp.maximum(_dot(xh_, k0n[:]) + _dot(agg, k1n[:])
                     + _dot(uh_t[:], kun[:]) + b1n[:], 0.0)
    x_a = _dot(h1, k2n[:]) + b2n[:]
    xa = _dot(jnp.sum(x_a, axis=0, keepdims=True), f[:]) * (1.0 / N_NODES)
    ea = _dot(jnp.sum(seg, axis=0, keepdims=True), f[:]) * (1.0 / N_EDGES)
    uh16 = uh_t[:, :H]
    ucat = jnp.concatenate([uh16, xa, ea], axis=1)
    hu = jnp.maximum(_dot(ucat, w1u[:]) + b1u[:, :H], 0.0)
    u_a = _dot(hu, w2u[:]) + b2u[:, :H]
    x2 = x_a * xh_
    x2_ref[:] = x2
    xpn_ref[:] = xp0r[:] + _dot(x2, kxr[:])
    u2 = u_a * uh16
    xa2 = _dot(jnp.sum(x2, axis=0, keepdims=True), f[:]) * (1.0 / N_NODES)
    ea2 = _dot(e2s[:], f[:]) * (1.0 / N_EDGES)
    gcat = jnp.concatenate([u2, xa2, ea2], axis=1)
    hg = jnp.maximum(_dot(gcat, w1g[:]) + b1g[:, :H], 0.0)
    u2p = _dot(hg, w2g[:]) + b2g[:, :H]
    u2_ref[:] = _dot(u2p, ft[:])
    hd = jnp.maximum(_dot(u2p, w1d[:]) + b1d[:, :H], 0.0)
    dec_ref[:] = _dot(_dot(hd, w2d[:]) + b2d[:, :H], ft[:])


def _att_node(parts, cnt, xhv, uh_t, e2s, xp0rv, kxrec, params):
    (W1n, b1n), (W2n, b2n) = params['att']['x']
    (W1u, b1u), (W2u, b2u) = params['att']['u']
    (W1g, b1g), (W2g, b2g) = params['agg']
    (W1d, b1d), (W2d, b2d) = params['dec']
    W2dp = jnp.zeros((H, H), _f32).at[:, :2].set(W2d)
    b2dp = jnp.zeros((H,), _f32).at[:2].set(b2d)
    f = jnp.tile(jnp.eye(H, dtype=_f32), (8, 1))
    ft = f.T
    return pl.pallas_call(
        _att_node_body,
        out_shape=[jax.ShapeDtypeStruct((NV, 128), _f32),
                   jax.ShapeDtypeStruct((NV, 128), _f32),
                   jax.ShapeDtypeStruct((1, 128), _f32),
                   jax.ShapeDtypeStruct((1, 128), _f32)],
    )(parts[0], parts[1], cnt[0], cnt[1], xhv, uh_t, e2s, xp0rv,
      _kron8(W1n[0:16]), _kron8(W1n[16:32]), _kron8(W1n[32:48]),
      _tile8(b1n), _kron8(W2n), _tile8(b2n), W1u, _tile8(b1u), W2u,
      _tile8(b2u), W1g, _tile8(b1g), W2g, _tile8(b2g), W1d, _tile8(b1d),
      W2dp, _tile8(b2dp), _kron8(kxrec), f, ft)


_NC, _NS = 2, 16
_NW = _NC * _NS
_EPW = N_EDGES // _NW
_GC = 2000
_NCH = _EPW // _GC
_NWB = 10
_NPT = N_NODES // _NWB

_sc_mesh = plsc.VectorSubcoreMesh(core_axis_name="c", subcore_axis_name="s")


_GCG = 1000
_NCHG = _EPW // _GCG


@functools.partial(
    pl.kernel,
    out_type=jax.ShapeDtypeStruct((N_EDGES, H), _f32),
    mesh=_sc_mesh,
    compiler_params=pltpu.CompilerParams(use_tc_tiling_on_sc=False),
    scratch_types=[pltpu.VMEM((_EPW,), jnp.int32),
                   pltpu.VMEM((_EPW,), jnp.int32),
                   pltpu.VMEM((_GCG, H), _f32),
                   pltpu.VMEM((_GCG, H), _f32),
                   pltpu.VMEM((_GCG, H), _f32),
                   pltpu.VMEM((_GCG, H), _f32),
                   pltpu.SemaphoreType.DMA,
                   pltpu.SemaphoreType.DMA,
                   pltpu.SemaphoreType.DMA],
)
def _sc_gather2(xp_hbm, src_hbm, dst_hbm, out_hbm, idx_s, idx_d,
                rs0, rd0, rs1, rd1, sem0, sem1, sem_i):
    wid = lax.axis_index("s") * _NC + lax.axis_index("c")
    base0 = pl.multiple_of(wid * _EPW, 8)
    hi1 = pltpu.async_copy(src_hbm.at[pl.ds(base0, _EPW)], idx_s, sem_i)
    hi2 = pltpu.async_copy(dst_hbm.at[pl.ds(base0, _EPW)], idx_d, sem_i)
    hi1.wait()
    hi2.wait()
    bufs = [(rs0, rd0, sem0), (rs1, rd1, sem1)]

    def issue(c):
        rs, rd, sem = bufs[c % 2]
        off = pl.multiple_of(c * _GCG, 8)
        h1 = pltpu.async_copy(xp_hbm.at[idx_s.at[pl.ds(off, _GCG)]], rs, sem)
        h2 = pltpu.async_copy(xp_hbm.at[idx_d.at[pl.ds(off, _GCG)]], rd, sem)
        return h1, h2

    def drain(c, hs):
        rs, rd, _ = bufs[c % 2]
        hs[0].wait()
        hs[1].wait()

        def sub8(r, carry):
            for k in range(8):
                i = r * 8 + k
                rd[i, :] = rd[i, :] - rs[i, :]
            return carry

        lax.fori_loop(0, _GCG // 8, sub8, 0)
        off = pl.multiple_of(c * _GCG, 8)
        pltpu.sync_copy(rd, out_hbm.at[pl.ds(base0 + off, _GCG)])

    hs = issue(0)
    for c in range(_NCHG):
        nxt = issue(c + 1) if c + 1 < _NCHG else None
        drain(c, hs)
        hs = nxt


def _scatter_body(e_hbm, dst_hbm, zeros_hbm, ones_hbm, out_hbm, idx0, rows0,
                  idx1, rows1, sem0, sem1, ones_v, accum, accum_c, gc,
                  with_counts):
    cid = lax.axis_index("c")
    sid = lax.axis_index("s")
    wid = sid * _NC + cid
    off = pl.multiple_of(sid * _NPT, 8)
    nch = _EPW // gc

    @pl.when(sid < _NWB)
    def _():
        pltpu.sync_copy(zeros_hbm.at[pl.ds(0, _NPT)],
                        accum.at[pl.ds(off, _NPT)])
        if with_counts:
            pltpu.sync_copy(zeros_hbm.at[pl.ds(0, _NPT)],
                            accum_c.at[pl.ds(off, _NPT)])

    if with_counts:
        pltpu.sync_copy(ones_hbm, ones_v)
    base0 = pl.multiple_of(wid * _EPW, 8)
    bufs = [(idx0, rows0, sem0), (idx1, rows1, sem1)]

    def issue(c):
        idx_v, rows_v, sem = bufs[c % 2]
        base = pl.multiple_of(base0 + c * gc, 8)
        h1 = pltpu.async_copy(dst_hbm.at[pl.ds(base, gc)], idx_v, sem)
        h2 = pltpu.async_copy(e_hbm.at[pl.ds(base, gc)], rows_v, sem)
        return h1, h2

    def drain(c, hs):
        idx_v, rows_v, _ = bufs[c % 2]
        hs[0].wait()
        hs[1].wait()
        pltpu.sync_copy(rows_v, accum.at[idx_v], add=True)
        if with_counts:
            pltpu.sync_copy(ones_v, accum_c.at[idx_v], add=True)

    hs = issue(0)
    plsc.subcore_barrier()
    for c in range(nch):
        nxt = issue(c + 1) if c + 1 < nch else None
        drain(c, hs)
        hs = nxt

    plsc.subcore_barrier()

    @pl.when(sid < _NWB)
    def _():
        dst_off = pl.multiple_of(cid * N_NODES + sid * _NPT, 8)
        pltpu.sync_copy(accum.at[pl.ds(off, _NPT)],
                        out_hbm.at[pl.ds(dst_off, _NPT)])
        if with_counts:
            cnt_off = pl.multiple_of(
                2 * N_NODES + cid * N_NODES + sid * _NPT, 8)
            pltpu.sync_copy(accum_c.at[pl.ds(off, _NPT)],
                            out_hbm.at[pl.ds(cnt_off, _NPT)])


@functools.partial(
    pl.kernel,
    out_type=jax.ShapeDtypeStruct((2 * N_NODES, H), _f32),
    mesh=_sc_mesh,
    compiler_params=pltpu.CompilerParams(use_tc_tiling_on_sc=False),
    scratch_types=[pltpu.VMEM((_GC,), jnp.int32),
                   pltpu.VMEM((_GC, H), _f32),
                   pltpu.VMEM((_GC,), jnp.int32),
                   pltpu.VMEM((_GC, H), _f32),
                   pltpu.SemaphoreType.DMA,
                   pltpu.SemaphoreType.DMA,
                   pltpu.VMEM_SHARED((N_NODES, H), _f32)],
)
def _sc_scatter(e_hbm, dst_hbm, zeros_hbm, out_hbm, idx0, rows0, idx1,
                rows1, sem0, sem1, accum):
    _scatter_body(e_hbm, dst_hbm, zeros_hbm, None, out_hbm, idx0, rows0,
                  idx1, rows1, sem0, sem1, None, accum, None, _GC, False)


_GCC = 1000


@functools.partial(
    pl.kernel,
    out_type=jax.ShapeDtypeStruct((4 * N_NODES, H), _f32),
    mesh=_sc_mesh,
    compiler_params=pltpu.CompilerParams(use_tc_tiling_on_sc=False),
    scratch_types=[pltpu.VMEM((_GCC,), jnp.int32),
                   pltpu.VMEM((_GCC, H), _f32),
                   pltpu.VMEM((_GCC,), jnp.int32),
                   pltpu.VMEM((_GCC, H), _f32),
                   pltpu.SemaphoreType.DMA,
                   pltpu.SemaphoreType.DMA,
                   pltpu.VMEM((_GCC, H), _f32),
                   pltpu.VMEM_SHARED((N_NODES, H), _f32),
                   pltpu.VMEM_SHARED((N_NODES, H), _f32)],
)
def _sc_scatter_counts(e_hbm, dst_hbm, zeros_hbm, ones_hbm, out_hbm, idx0,
                       rows0, idx1, rows1, sem0, sem1, ones_v, accum,
                       accum_c):
    _scatter_body(e_hbm, dst_hbm, zeros_hbm, ones_hbm, out_hbm, idx0, rows0,
                  idx1, rows1, sem0, sem1, ones_v, accum, accum_c, _GCC,
                  True)


def _gather_diff(xp16, src, dst):
    return jnp.reshape(_sc_gather2(xp16, src, dst), (EV, 128))


def _scatter_parts(e16, dst):
    zeros = jnp.zeros((_NPT, H), _f32)
    return jnp.reshape(_sc_scatter(e16, dst, zeros), (2, NV, 128))


def _scatter_parts_counts(e16, dst):
    zeros = jnp.zeros((_NPT, H), _f32)
    ones = jnp.ones((_GCC, H), _f32)
    return jnp.reshape(_sc_scatter_counts(e16, dst, zeros, ones),
                       (4, NV, 128))


def kernel(x1, edge_index1, e1, u1, batch1, x2, edge_index2, e2, u2, batch2,
           params):
    del batch1, batch2
    W1r_e = params['rec']['e'][0][0]
    W1a_e = params['att']['e'][0][0]
    kx_rec = W1r_e[160:176]
    kx_att = W1a_e[16:32]
    zero_nv = jnp.zeros((NV, 128), _f32)

    def prep(x, e, ei, u):
        src = ei[0]
        dst = ei[1]
        ev = jnp.reshape(e, (EV, 128))
        eh0v, e0v = _prep_edges(ev, params)
        xh0, xp0, xp1, xn0, uh0_t = _prep_nodes(x, _tile8(u), params)
        return dict(src=src, dst=dst, e0v=e0v, cnt=None,
                    xp0v=jnp.reshape(xp0, (NV, 128)),
                    xn0v=jnp.reshape(xn0, (NV, 128)),
                    ehv=eh0v, xhv=jnp.reshape(xh0, (NV, 128)),
                    xpv=jnp.reshape(xp1, (NV, 128)), uh_t=uh0_t)

    g1 = prep(x1, e1, edge_index1, u1)
    g2 = prep(x2, e2, edge_index2, u2)
    u1_t = _tile8(u1)
    u2_t = _tile8(u2)

    def processing(g, u_t, shared_t, first):
        xhv, ehv, xpv, uh_t = g['xhv'], g['ehv'], g['xpv'], g['uh_t']
        gn = dict(g)
        for inner in range(2):
            dxv = _gather_diff(jnp.reshape(xpv, (N_NODES, H)),
                               g['src'], g['dst'])
            env = _rec_edge(g['e0v'], ehv, dxv, u_t, shared_t, params)
            if first and inner == 0:
                parts4 = _scatter_parts_counts(jnp.reshape(env, (N_EDGES, H)),
                                               g['dst'])
                parts = parts4[:2]
                gn['cnt'] = parts4[2:]
            else:
                parts = _scatter_parts(jnp.reshape(env, (N_EDGES, H)),
                                       g['dst'])
            kxn = kx_rec if inner == 0 else kx_att
            xp0n = g['xp0v'] if inner == 0 else zero_nv
            xhv, xpv, uh_t = _rec_node(parts, gn['cnt'], g['xn0v'], xhv,
                                       u_t, shared_t, xp0n, kxn, params)
            ehv = env
        dxv = _gather_diff(jnp.reshape(xpv, (N_NODES, H)),
                           g['src'], g['dst'])
        eav, e2v, e2s = _att_edge(ehv, dxv, uh_t, params)
        parts = _scatter_parts(jnp.reshape(eav, (N_EDGES, H)), g['dst'])
        x2v, xpnv, u2_t, dec_t = _att_node(parts, gn['cnt'], xhv, uh_t, e2s,
                                           g['xp0v'], kx_rec, params)
        gn['xhv'], gn['ehv'], gn['xpv'], gn['uh_t'] = x2v, e2v, xpnv, u2_t
        return gn, dec_t

    outs = []
    for p in range(2):
        g1, _dec1 = processing(g1, u1_t, g2['uh_t'], p == 0)
        g2, dec2 = processing(g2, u2_t, g1['uh_t'], p == 0)
        outs.append(dec2[:, :2])
    return jnp.stack(outs)


# speedup vs baseline: 3.9049x; 1.0016x over previous
import functools

import jax
import jax.numpy as jnp
from jax import lax
from jax.experimental import pallas as pl
from jax.experimental.pallas import tpu as pltpu
from jax.experimental.pallas import tpu_sc as plsc

N_NODES = 10000
N_EDGES = 320000
EV = N_EDGES // 8
NV = N_NODES // 8
BLK = 5000
H = 16

_f32 = jnp.float32


def _kron8(W):
    return jnp.kron(jnp.eye(8, dtype=_f32), W.astype(_f32))


def _tile8(v):
    return jnp.tile(jnp.reshape(v, (1, H)), (1, 8))


def _dot(a, b):
    return jnp.dot(a, b, preferred_element_type=_f32)


def _prep_edges_body(e_ref, ke1, be1, ke2, be2, k0, b1e, eh0_ref, e0_ref):
    e = e_ref[:]
    h = jnp.maximum(_dot(e, ke1[:]) + be1[:], 0.0)
    eh0_ref[:] = _dot(h, ke2[:]) + be2[:]
    e0_ref[:] = _dot(e, k0[:]) + b1e[:]


def _prep_edges(ev, params):
    (W1, b1), (W2, b2) = params['enc']['e']
    W1r, b1r = params['rec']['e'][0]
    big = pl.BlockSpec((BLK, 128), lambda i: (i, 0))
    w = pl.BlockSpec((128, 128), lambda i: (0, 0))
    s = pl.BlockSpec((1, 128), lambda i: (0, 0))
    return pl.pallas_call(
        _prep_edges_body,
        grid=(EV // BLK,),
        in_specs=[big, w, s, w, s, w, s],
        out_specs=[big, big],
        out_shape=[jax.ShapeDtypeStruct((EV, 128), _f32)] * 2,
    )(ev, _kron8(W1), _tile8(b1), _kron8(W2), _tile8(b2),
      _kron8(W1r[0:16]), _tile8(b1r))


def _prep_nodes_body(x_ref, u_ref, w1x, b1x, w2x, b2x, wxp, wxn,
                     wxp1, w1u, b1u, w2u, b2u, ft,
                     xh0_ref, xp0_ref, xp1_ref, xn0_ref, uh0_ref):
    x = x_ref[:]
    h = jnp.maximum(_dot(x, w1x[:]) + b1x[:], 0.0)
    xh0 = _dot(h, w2x[:]) + b2x[:]
    xh0_ref[:] = xh0
    xp0 = _dot(x, wxp[:])
    xp0_ref[:] = xp0
    xp1_ref[:] = xp0 + _dot(xh0, wxp1[:])
    xn0_ref[:] = _dot(x, wxn[:])

    @pl.when(pl.program_id(0) == 0)
    def _():
        u16 = u_ref[:, :H]
        hu = jnp.maximum(_dot(u16, w1u[:]) + b1u[:], 0.0)
        uh0_ref[:] = _dot(_dot(hu, w2u[:]) + b2u[:], ft[:])


_NBLK = 2000


def _prep_nodes(x, u_t, params):
    (W1x, b1x), (W2x, b2x) = params['enc']['x']
    (W1u, b1u), (W2u, b2u) = params['enc']['u']
    W1r, _ = params['rec']['e'][0]
    W1n, _ = params['rec']['x'][0]
    ft = jnp.tile(jnp.eye(H, dtype=_f32), (1, 8))
    nb = pl.BlockSpec((_NBLK, 128), lambda i: (i, 0))
    nh = pl.BlockSpec((_NBLK, H), lambda i: (i, 0))
    w = lambda shp: pl.BlockSpec(shp, lambda i: tuple(0 for _ in shp))
    outs = pl.pallas_call(
        _prep_nodes_body,
        grid=(N_NODES // _NBLK,),
        in_specs=[nb, w((1, 128)), w((128, H)), w((1, H)), w((H, H)),
                  w((1, H)), w((128, H)), w((128, H)), w((H, H)),
                  w((H, H)), w((1, H)), w((H, H)), w((1, H)), w((H, 128))],
        out_specs=[nh, nh, nh, nh, w((1, 128))],
        out_shape=[jax.ShapeDtypeStruct((N_NODES, H), _f32)] * 4
        + [jax.ShapeDtypeStruct((1, 128), _f32)],
    )(x, u_t, W1x, jnp.reshape(b1x, (1, H)), W2x,
      jnp.reshape(b2x, (1, H)), W1r[32:160], W1n[0:128], W1r[160:176],
      W1u, jnp.reshape(b1u, (1, H)), W2u, jnp.reshape(b2u, (1, H)), ft)
    return outs


def _edge_mlp_body(e0_ref, eh_ref, dx_ref, u_t, s_t, ku, ks, k16,
                   k2, b2, out_ref):
    ut = _dot(u_t[:], ku[:]) + _dot(s_t[:], ks[:])
    h1 = jnp.maximum(e0_ref[:] + _dot(eh_ref[:], k16[:]) + dx_ref[:] + ut,
                     0.0)
    out_ref[:] = _dot(h1, k2[:]) + b2[:]


def _rec_edge(e0v, ehv, dxv, u_t, s_t, params):
    (W1, _), (W2, b2) = params['rec']['e']
    big = pl.BlockSpec((BLK, 128), lambda i: (i, 0))
    w = pl.BlockSpec((128, 128), lambda i: (0, 0))
    s = pl.BlockSpec((1, 128), lambda i: (0, 0))
    return pl.pallas_call(
        _edge_mlp_body,
        grid=(EV // BLK,),
        in_specs=[big, big, big, s, s, w, w, w, w, s],
        out_specs=big,
        out_shape=jax.ShapeDtypeStruct((EV, 128), _f32),
    )(e0v, ehv, dxv, u_t, s_t, _kron8(W1[176:192]),
      _kron8(W1[192:208]), _kron8(W1[16:32]), _kron8(W2), _tile8(b2))


def _rec_node_body(p0, p1, c0, c1, xn0, xh, u_t, s_t, xp0n, kb, kc, kun, ksn,
                   b1n, k2n, b2n, w1u, b1u, w2u, b2u, kxn, f, ft,
                   xnew_ref, xpn_ref, unew_ref):
    seg = p0[:] + p1[:]
    agg = seg / jnp.maximum(c0[:] + c1[:], 1.0)
    un = _dot(u_t[:], kun[:]) + _dot(s_t[:], ksn[:])
    h1 = jnp.maximum(xn0[:] + _dot(xh[:], kb[:]) + _dot(agg, kc[:]) + un
                     + b1n[:], 0.0)
    xnew = _dot(h1, k2n[:]) + b2n[:]
    xnew_ref[:] = xnew
    xpn_ref[:] = xp0n[:] + _dot(xnew, kxn[:])
    xa = _dot(jnp.sum(xnew, axis=0, keepdims=True), f[:]) * (1.0 / N_NODES)
    ea = _dot(jnp.sum(seg, axis=0, keepdims=True), f[:]) * (1.0 / N_EDGES)
    ucat = jnp.concatenate([u_t[:, :H], s_t[:, :H], xa, ea], axis=1)
    hu = jnp.maximum(_dot(ucat, w1u[:]) + b1u[:, :H], 0.0)
    unew_ref[:] = _dot(_dot(hu, w2u[:]) + b2u[:, :H], ft[:])


def _rec_node(parts, cnt, xn0v, xhv, u_t, s_t, xp0nextv, kxnext, params):
    (W1n, b1n), (W2n, b2n) = params['rec']['x']
    (W1u, b1u), (W2u, b2u) = params['rec']['u']
    f = jnp.tile(jnp.eye(H, dtype=_f32), (8, 1))
    ft = f.T
    return pl.pallas_call(
        _rec_node_body,
        out_shape=[jax.ShapeDtypeStruct((NV, 128), _f32),
                   jax.ShapeDtypeStruct((NV, 128), _f32),
                   jax.ShapeDtypeStruct((1, 128), _f32)],
    )(parts[0], parts[1], cnt[0], cnt[1], xn0v, xhv, u_t, s_t, xp0nextv,
      _kron8(W1n[128:144]), _kron8(W1n[144:160]), _kron8(W1n[160:176]),
      _kron8(W1n[176:192]), _tile8(b1n), _kron8(W2n), _tile8(b2n),
      W1u, _tile8(b1u), W2u, _tile8(b2u), _kron8(kxnext), f, ft)


def _att_edge_body(eh_ref, dx_ref, uh_t, ku, k0, k2, b1, b2,
                   ea_ref, e2_ref, e2s_ref):
    ut = _dot(uh_t[:], ku[:]) + b1[:]
    eh = eh_ref[:]
    h1 = jnp.maximum(_dot(eh, k0[:]) + dx_ref[:] + ut, 0.0)
    ea = _dot(h1, k2[:]) + b2[:]
    ea_ref[:] = ea
    e2 = ea * eh
    e2_ref[:] = e2

    @pl.when(pl.program_id(0) == 0)
    def _():
        e2s_ref[:] = jnp.zeros_like(e2s_ref)

    e2s_ref[:] += jnp.sum(e2, axis=0, keepdims=True)


def _att_edge(ehv, dxv, uh_t, params):
    (W1, b1), (W2, b2) = params['att']['e']
    big = pl.BlockSpec((BLK, 128), lambda i: (i, 0))
    w = pl.BlockSpec((128, 128), lambda i: (0, 0))
    s = pl.BlockSpec((1, 128), lambda i: (0, 0))
    return pl.pallas_call(
        _att_edge_body,
        grid=(EV // BLK,),
        in_specs=[big, big, s, w, w, w, s, s],
        out_specs=[big, big, s],
        out_shape=[jax.ShapeDtypeStruct((EV, 128), _f32)] * 2
        + [jax.ShapeDtypeStruct((1, 128), _f32)],
    )(ehv, dxv, uh_t, _kron8(W1[32:48]), _kron8(W1[0:16]), _kron8(W2),
      _tile8(b1), _tile8(b2))


def _att_node_body(p0, p1, c0, c1, xh, uh_t, e2s, xp0r, k0n, k1n, kun, b1n,
                   k2n, b2n, w1u, b1u, w2u, b2u, w1g, b1g, w2g, b2g,
                   w1d, b1d, w2d, b2d, kxr, f, ft,
                   x2_ref, xpn_ref, u2_ref, dec_ref):
    seg = p0[:] + p1[:]
    agg = seg / jnp.maximum(c0[:] + c1[:], 1.0)
    xh_ = xh[:]
    h1 = jnp.maximum(_dot(xh_, k0n[:]) + _dot(agg, k1n[:])
                     + _dot(uh_t[:], kun[:]) + b1n[:], 0.0)
    x_a = _dot(h1, k2n[:]) + b2n[:]
    xa = _dot(jnp.sum(x_a, axis=0, keepdims=True), f[:]) * (1.0 / N_NODES)
    ea = _dot(jnp.sum(seg, axis=0, keepdims=True), f[:]) * (1.0 / N_EDGES)
    uh16 = uh_t[:, :H]
    ucat = jnp.concatenate([uh16, xa, ea], axis=1)
    hu = jnp.maximum(_dot(ucat, w1u[:]) + b1u[:, :H], 0.0)
    u_a = _dot(hu, w2u[:]) + b2u[:, :H]
    x2 = x_a * xh_
    x2_ref[:] = x2
    xpn_ref[:] = xp0r[:] + _dot(x2, kxr[:])
    u2 = u_a * uh16
    xa2 = _dot(jnp.sum(x2, axis=0, keepdims=True), f[:]) * (1.0 / N_NODES)
    ea2 = _dot(e2s[:], f[:]) * (1.0 / N_EDGES)
    gcat = jnp.concatenate([u2, xa2, ea2], axis=1)
    hg = jnp.maximum(_dot(gcat, w1g[:]) + b1g[:, :H], 0.0)
    u2p = _dot(hg, w2g[:]) + b2g[:, :H]
    u2_ref[:] = _dot(u2p, ft[:])
    hd = jnp.maximum(_dot(u2p, w1d[:]) + b1d[:, :H], 0.0)
    dec_ref[:] = _dot(_dot(hd, w2d[:]) + b2d[:, :H], ft[:])


def _att_node(parts, cnt, xhv, uh_t, e2s, xp0rv, kxrec, params):
    (W1n, b1n), (W2n, b2n) = params['att']['x']
    (W1u, b1u), (W2u, b2u) = params['att']['u']
    (W1g, b1g), (W2g, b2g) = params['agg']
    (W1d, b1d), (W2d, b2d) = params['dec']
    W2dp = jnp.zeros((H, H), _f32).at[:, :2].set(W2d)
    b2dp = jnp.zeros((H,), _f32).at[:2].set(b2d)
    f = jnp.tile(jnp.eye(H, dtype=_f32), (8, 1))
    ft = f.T
    return pl.pallas_call(
        _att_node_body,
        out_shape=[jax.ShapeDtypeStruct((NV, 128), _f32),
                   jax.ShapeDtypeStruct((NV, 128), _f32),
                   jax.ShapeDtypeStruct((1, 128), _f32),
                   jax.ShapeDtypeStruct((1, 128), _f32)],
    )(parts[0], parts[1], cnt[0], cnt[1], xhv, uh_t, e2s, xp0rv,
      _kron8(W1n[0:16]), _kron8(W1n[16:32]), _kron8(W1n[32:48]),
      _tile8(b1n), _kron8(W2n), _tile8(b2n), W1u, _tile8(b1u), W2u,
      _tile8(b2u), W1g, _tile8(b1g), W2g, _tile8(b2g), W1d, _tile8(b1d),
      W2dp, _tile8(b2dp), _kron8(kxrec), f, ft)


_NC, _NS = 2, 16
_NW = _NC * _NS
_EPW = N_EDGES // _NW
_GC = 2000
_NCH = _EPW // _GC
_NWB = 10
_NPT = N_NODES // _NWB

_sc_mesh = plsc.VectorSubcoreMesh(core_axis_name="c", subcore_axis_name="s")


_GCG = 1000
_NCHG = _EPW // _GCG


@functools.partial(
    pl.kernel,
    out_type=jax.ShapeDtypeStruct((N_EDGES, H), _f32),
    mesh=_sc_mesh,
    compiler_params=pltpu.CompilerParams(use_tc_tiling_on_sc=False),
    scratch_types=[pltpu.VMEM((_EPW,), jnp.int32),
                   pltpu.VMEM((_EPW,), jnp.int32),
                   pltpu.VMEM((_GCG, H), _f32),
                   pltpu.VMEM((_GCG, H), _f32),
                   pltpu.VMEM((_GCG, H), _f32),
                   pltpu.VMEM((_GCG, H), _f32),
                   pltpu.SemaphoreType.DMA,
                   pltpu.SemaphoreType.DMA,
                   pltpu.SemaphoreType.DMA],
)
def _sc_gather2(xp_hbm, src_hbm, dst_hbm, out_hbm, idx_s, idx_d,
                rs0, rd0, rs1, rd1, sem0, sem1, sem_i):
    wid = lax.axis_index("s") * _NC + lax.axis_index("c")
    base0 = pl.multiple_of(wid * _EPW, 8)
    hi1 = pltpu.async_copy(src_hbm.at[pl.ds(base0, _EPW)], idx_s, sem_i)
    hi2 = pltpu.async_copy(dst_hbm.at[pl.ds(base0, _EPW)], idx_d, sem_i)
    hi1.wait()
    hi2.wait()
    bufs = [(rs0, rd0, sem0), (rs1, rd1, sem1)]

    def issue(c):
        rs, rd, sem = bufs[c % 2]
        off = pl.multiple_of(c * _GCG, 8)
        h1 = pltpu.async_copy(xp_hbm.at[idx_s.at[pl.ds(off, _GCG)]], rs, sem)
        h2 = pltpu.async_copy(xp_hbm.at[idx_d.at[pl.ds(off, _GCG)]], rd, sem)
        return h1, h2

    def drain(c, hs):
        rs, rd, _ = bufs[c % 2]
        hs[0].wait()
        hs[1].wait()

        def sub8(r, carry):
            for k in range(8):
                i = r * 8 + k
                rd[i, :] = rd[i, :] - rs[i, :]
            return carry

        lax.fori_loop(0, _GCG // 8, sub8, 0)
        off = pl.multiple_of(c * _GCG, 8)
        pltpu.sync_copy(rd, out_hbm.at[pl.ds(base0 + off, _GCG)])

    hs = issue(0)
    for c in range(_NCHG):
        nxt = issue(c + 1) if c + 1 < _NCHG else None
        drain(c, hs)
        hs = nxt


def _scatter_body(e_hbm, dst_hbm, zeros_hbm, ones_hbm, out_hbm, idx0, rows0,
                  idx1, rows1, sem0, sem1, ones_v, accum, accum_c, gc,
                  with_counts):
    cid = lax.axis_index("c")
    sid = lax.axis_index("s")
    wid = sid * _NC + cid
    off = pl.multiple_of(sid * _NPT, 8)
    nch = _EPW // gc

    @pl.when(sid < _NWB)
    def _():
        pltpu.sync_copy(zeros_hbm.at[pl.ds(0, _NPT)],
                        accum.at[pl.ds(off, _NPT)])
        if with_counts:
            pltpu.sync_copy(zeros_hbm.at[pl.ds(0, _NPT)],
                            accum_c.at[pl.ds(off, _NPT)])

    if with_counts:
        pltpu.sync_copy(ones_hbm, ones_v)
    base0 = pl.multiple_of(wid * _EPW, 8)
    bufs = [(idx0, rows0, sem0), (idx1, rows1, sem1)]

    def issue(c):
        idx_v, rows_v, sem = bufs[c % 2]
        base = pl.multiple_of(base0 + c * gc, 8)
        h1 = pltpu.async_copy(dst_hbm.at[pl.ds(base, gc)], idx_v, sem)
        h2 = pltpu.async_copy(e_hbm.at[pl.ds(base, gc)], rows_v, sem)
        return h1, h2

    def drain(c, hs):
        idx_v, rows_v, _ = bufs[c % 2]
        hs[0].wait()
        hs[1].wait()
        pltpu.sync_copy(rows_v, accum.at[idx_v], add=True)
        if with_counts:
            pltpu.sync_copy(ones_v, accum_c.at[idx_v], add=True)

    hs = issue(0)
    plsc.subcore_barrier()
    for c in range(nch):
        nxt = issue(c + 1) if c + 1 < nch else None
        drain(c, hs)
        hs = nxt

    plsc.subcore_barrier()

    @pl.when(sid < _NWB)
    def _():
        dst_off = pl.multiple_of(cid * N_NODES + sid * _NPT, 8)
        pltpu.sync_copy(accum.at[pl.ds(off, _NPT)],
                        out_hbm.at[pl.ds(dst_off, _NPT)])
        if with_counts:
            cnt_off = pl.multiple_of(
                2 * N_NODES + cid * N_NODES + sid * _NPT, 8)
            pltpu.sync_copy(accum_c.at[pl.ds(off, _NPT)],
                            out_hbm.at[pl.ds(cnt_off, _NPT)])


@functools.partial(
    pl.kernel,
    out_type=jax.ShapeDtypeStruct((2 * N_NODES, H), _f32),
    mesh=_sc_mesh,
    compiler_params=pltpu.CompilerParams(use_tc_tiling_on_sc=False),
    scratch_types=[pltpu.VMEM((_GC,), jnp.int32),
                   pltpu.VMEM((_GC, H), _f32),
                   pltpu.VMEM((_GC,), jnp.int32),
                   pltpu.VMEM((_GC, H), _f32),
                   pltpu.SemaphoreType.DMA,
                   pltpu.SemaphoreType.DMA,
                   pltpu.VMEM_SHARED((N_NODES, H), _f32)],
)
def _sc_scatter(e_hbm, dst_hbm, zeros_hbm, out_hbm, idx0, rows0, idx1,
                rows1, sem0, sem1, accum):
    _scatter_body(e_hbm, dst_hbm, zeros_hbm, None, out_hbm, idx0, rows0,
                  idx1, rows1, sem0, sem1, None, accum, None, _GC, False)


_GCC = 1000


@functools.partial(
    pl.kernel,
    out_type=jax.ShapeDtypeStruct((4 * N_NODES, H), _f32),
    mesh=_sc_mesh,
    compiler_params=pltpu.CompilerParams(use_tc_tiling_on_sc=False),
    scratch_types=[pltpu.VMEM((_GCC,), jnp.int32),
                   pltpu.VMEM((_GCC, H), _f32),
                   pltpu.VMEM((_GCC,), jnp.int32),
                   pltpu.VMEM((_GCC, H), _f32),
                   pltpu.SemaphoreType.DMA,
                   pltpu.SemaphoreType.DMA,
                   pltpu.VMEM((_GCC, H), _f32),
                   pltpu.VMEM_SHARED((N_NODES, H), _f32),
                   pltpu.VMEM_SHARED((N_NODES, H), _f32)],
)
def _sc_scatter_counts(e_hbm, dst_hbm, zeros_hbm, ones_hbm, out_hbm, idx0,
                       rows0, idx1, rows1, sem0, sem1, ones_v, accum,
                       accum_c):
    _scatter_body(e_hbm, dst_hbm, zeros_hbm, ones_hbm, out_hbm, idx0, rows0,
                  idx1, rows1, sem0, sem1, ones_v, accum, accum_c, _GCC,
                  True)


def _gather_diff(xp16, src, dst):
    return jnp.reshape(_sc_gather2(xp16, src, dst), (EV, 128))


def _scatter_parts(e16, dst):
    zeros = jnp.zeros((_NPT, H), _f32)
    return jnp.reshape(_sc_scatter(e16, dst, zeros), (2, NV, 128))


def _scatter_parts_counts(e16, dst):
    zeros = jnp.zeros((_NPT, H), _f32)
    ones = jnp.ones((_GCC, H), _f32)
    return jnp.reshape(_sc_scatter_counts(e16, dst, zeros, ones),
                       (4, NV, 128))


def kernel(x1, edge_index1, e1, u1, batch1, x2, edge_index2, e2, u2, batch2,
           params):
    del batch1, batch2
    W1r_e = params['rec']['e'][0][0]
    W1a_e = params['att']['e'][0][0]
    kx_rec = W1r_e[160:176]
    kx_att = W1a_e[16:32]
    zero_nv = jnp.zeros((NV, 128), _f32)

    def prep(x, e, ei, u):
        src = ei[0]
        dst = ei[1]
        ev = jnp.reshape(e, (EV, 128))
        eh0v, e0v = _prep_edges(ev, params)
        xh0, xp0, xp1, xn0, uh0_t = _prep_nodes(x, _tile8(u), params)
        return dict(src=src, dst=dst, e0v=e0v, cnt=None,
                    xp0v=jnp.reshape(xp0, (NV, 128)),
                    xn0v=jnp.reshape(xn0, (NV, 128)),
                    ehv=eh0v, xhv=jnp.reshape(xh0, (NV, 128)),
                    xpv=jnp.reshape(xp1, (NV, 128)), uh_t=uh0_t)

    g1 = prep(x1, e1, edge_index1, u1)
    g2 = prep(x2, e2, edge_index2, u2)
    u1_t = _tile8(u1)
    u2_t = _tile8(u2)

    def processing(g, u_t, shared_t, first):
        xhv, ehv, xpv, uh_t = g['xhv'], g['ehv'], g['xpv'], g['uh_t']
        gn = dict(g)
        for inner in range(2):
            dxv = _gather_diff(jnp.reshape(xpv, (N_NODES, H)),
                               g['src'], g['dst'])
            env = _rec_edge(g['e0v'], ehv, dxv, u_t, shared_t, params)
            if first and inner == 0:
                parts4 = _scatter_parts_counts(jnp.reshape(env, (N_EDGES, H)),
                                               g['dst'])
                parts = parts4[:2]
                gn['cnt'] = parts4[2:]
            else:
                parts = _scatter_parts(jnp.reshape(env, (N_EDGES, H)),
                                       g['dst'])
            kxn = kx_rec if inner == 0 else kx_att
            xp0n = g['xp0v'] if inner == 0 else zero_nv
            xhv, xpv, uh_t = _rec_node(parts, gn['cnt'], g['xn0v'], xhv,
                                       u_t, shared_t, xp0n, kxn, params)
            ehv = env
        dxv = _gather_diff(jnp.reshape(xpv, (N_NODES, H)),
                           g['src'], g['dst'])
        eav, e2v, e2s = _att_edge(ehv, dxv, uh_t, params)
        parts = _scatter_parts(jnp.reshape(eav, (N_EDGES, H)), g['dst'])
        x2v, xpnv, u2_t, dec_t = _att_node(parts, gn['cnt'], xhv, uh_t, e2s,
                                           g['xp0v'], kx_rec, params)
        gn['xhv'], gn['ehv'], gn['xpv'], gn['uh_t'] = x2v, e2v, xpnv, u2_t
        return gn, dec_t

    outs = []
    for p in range(2):
        g1, _dec1 = processing(g1, u1_t, g2['uh_t'], p == 0)
        g2, dec2 = processing(g2, u2_t, g1['uh_t'], p == 0)
        outs.append(dec2[:, :2])
    return jnp.stack(outs)
